# baseline (speedup 1.0000x reference)
import functools

import jax
import jax.numpy as jnp
from jax import lax
from jax.experimental import pallas as pl
from jax.experimental.pallas import tpu as pltpu
from jax.experimental.pallas import tpu_sc as plsc

VOCAB = 1000
VB = VOCAB // 8
NC = 2
NS = 16
NW = NC * NS
KMAX = -(-VB // NW)


def _body(B, L, idxT_hbm, tableT_hbm, out_hbm, tcols, irow0, irow1,
          obuf0, obuf1, isem0, isem1, osem0, osem1):
    c = lax.axis_index("c")
    s = lax.axis_index("s")
    w = s * NC + c
    nbb = B // 128
    irows = (irow0, irow1)
    isems = (isem0, isem1)
    obufs = (obuf0, obuf1)
    osems = (osem0, osem1)

    def build_block(irow, obuf):
        def bb_body(bb, carry):
            base = bb * 128
            ivs = [irow[pl.ds(base + gg * 16, 16)] for gg in range(8)]
            prev = None
            for gg in range(8):
                cur = []
                for vr in range(8):
                    cur.append(plsc.load_gather(tcols.at[vr], [ivs[gg]]))
                    if prev is not None:
                        obuf[bb, vr, pl.ds((gg - 1) * 16, 16)] = prev[vr]
                prev = cur
            for vr in range(8):
                obuf[bb, vr, pl.ds(7 * 16, 16)] = prev[vr]
            return carry
        lax.fori_loop(0, nbb, bb_body, 0)

    for kk in range(KMAX):
        vb = w + NW * kk

        @pl.when(vb < VB)
        def _():
            pltpu.sync_copy(tableT_hbm.at[pl.ds(vb * 8, 8)], tcols)
            pltpu.async_copy(idxT_hbm.at[0], irow0, isem0)

            def l_step(l, carry):
                for p in range(2):
                    li = 2 * l + p
                    pltpu.make_async_copy(
                        idxT_hbm.at[li], irows[p], isems[p]).wait()

                    @pl.when(li + 1 < L)
                    def _():
                        pltpu.async_copy(
                            idxT_hbm.at[li + 1], irows[1 - p], isems[1 - p])

                    @pl.when(li >= 2)
                    def _():
                        pltpu.make_async_copy(
                            obufs[p], out_hbm.at[li - 2, vb], osems[p]).wait()

                    build_block(irows[p], obufs[p])
                    pltpu.async_copy(
                        obufs[p], out_hbm.at[li, vb], osems[p])
                return carry

            lax.fori_loop(0, L // 2, l_step, 0)
            pltpu.make_async_copy(
                obufs[0], out_hbm.at[L - 2, vb], osems[0]).wait()
            pltpu.make_async_copy(
                obufs[1], out_hbm.at[L - 1, vb], osems[1]).wait()


def kernel(idx, logits_table):
    B, L = idx.shape
    assert B % (128 * NW) == 0 and L % 2 == 0 and VOCAB % 8 == 0
    idx_t = idx.T.astype(jnp.int32)
    table_t = logits_table.T

    mesh = plsc.VectorSubcoreMesh(core_axis_name="c", subcore_axis_name="s")
    k = pl.kernel(
        functools.partial(_body, B, L),
        out_type=jax.ShapeDtypeStruct((L, VB, B // 128, 8, 128), jnp.float32),
        mesh=mesh,
        scratch_types=[
            pltpu.VMEM((8, VOCAB), jnp.float32),
            pltpu.VMEM((B,), jnp.int32),
            pltpu.VMEM((B,), jnp.int32),
            pltpu.VMEM((B // 128, 8, 128), jnp.float32),
            pltpu.VMEM((B // 128, 8, 128), jnp.float32),
            pltpu.SemaphoreType.DMA,
            pltpu.SemaphoreType.DMA,
            pltpu.SemaphoreType.DMA,
            pltpu.SemaphoreType.DMA,
        ],
        compiler_params=pltpu.CompilerParams(use_tc_tiling_on_sc=False,
                                             needs_layout_passes=False),
    )
    out5 = k(idx_t, table_t)
    t = out5.transpose(2, 4, 0, 1, 3)
    return t.reshape(B, L, VOCAB)


# speedup vs baseline: 1.1197x; 1.1197x over previous
import functools

import jax
import jax.numpy as jnp
from jax import lax
from jax.experimental import pallas as pl
from jax.experimental.pallas import tpu as pltpu
from jax.experimental.pallas import tpu_sc as plsc

VOCAB = 1000
VB = VOCAB // 8
NC = 2
NS = 16
NW = NC * NS
KMAX = -(-VB // NW)


def _body(B, L, idxT_hbm, tableT_hbm, out_hbm, tcols, irow0, irow1,
          obuf0, obuf1, isem0, isem1, osem0, osem1):
    c = lax.axis_index("c")
    s = lax.axis_index("s")
    w = s * NC + c
    nbb = B // 128
    irows = (irow0, irow1)
    isems = (isem0, isem1)
    obufs = (obuf0, obuf1)
    osems = (osem0, osem1)

    def build_block(kk, irow, obuf):
        def bb_body(bb, carry):
            base = bb * 128
            ivs = [irow[pl.ds(base + gg * 16, 16)] for gg in range(8)]
            prev = None
            for gg in range(8):
                cur = []
                for vr in range(8):
                    cur.append(plsc.load_gather(tcols.at[kk, vr], [ivs[gg]]))
                    if prev is not None:
                        obuf[bb, vr, pl.ds((gg - 1) * 16, 16)] = prev[vr]
                prev = cur
            for vr in range(8):
                obuf[bb, vr, pl.ds(7 * 16, 16)] = prev[vr]
            return carry
        lax.fori_loop(0, nbb, bb_body, 0)

    for kk in range(KMAX):
        vb = w + NW * kk

        @pl.when(vb < VB)
        def _():
            pltpu.sync_copy(tableT_hbm.at[pl.ds(vb * 8, 8)], tcols.at[kk])

    pltpu.async_copy(idxT_hbm.at[0], irow0, isem0)

    def l_step(h, carry):
        for lp in range(2):
            l = 2 * h + lp
            pltpu.make_async_copy(idxT_hbm.at[l], irows[lp], isems[lp]).wait()

            @pl.when(l + 1 < L)
            def _():
                pltpu.async_copy(
                    idxT_hbm.at[l + 1], irows[1 - lp], isems[1 - lp])

            for kk in range(KMAX):
                vb = w + NW * kk
                p = kk % 2

                @pl.when(vb < VB)
                def _():
                    if kk >= 2:
                        pltpu.make_async_copy(
                            obufs[p], out_hbm.at[0, 0], osems[p]).wait()
                    else:
                        @pl.when(l >= 1)
                        def _():
                            pltpu.make_async_copy(
                                obufs[p], out_hbm.at[0, 0], osems[p]).wait()

                    build_block(kk, irows[lp], obufs[p])
                    pltpu.async_copy(obufs[p], out_hbm.at[l, vb], osems[p])
        return carry

    lax.fori_loop(0, L // 2, l_step, 0)
    pltpu.make_async_copy(obufs[0], out_hbm.at[0, 0], osems[0]).wait()
    pltpu.make_async_copy(obufs[1], out_hbm.at[0, 0], osems[1]).wait()


def kernel(idx, logits_table):
    B, L = idx.shape
    assert B % (128 * NW) == 0 and VB >= 2 * NW
    idx_t = idx.T.astype(jnp.int32)
    table_t = logits_table.T

    mesh = plsc.VectorSubcoreMesh(core_axis_name="c", subcore_axis_name="s")
    k = pl.kernel(
        functools.partial(_body, B, L),
        out_type=jax.ShapeDtypeStruct((L, VB, B // 128, 8, 128), jnp.float32),
        mesh=mesh,
        scratch_types=[
            pltpu.VMEM((KMAX, 8, VOCAB), jnp.float32),
            pltpu.VMEM((B,), jnp.int32),
            pltpu.VMEM((B,), jnp.int32),
            pltpu.VMEM((B // 128, 8, 128), jnp.float32),
            pltpu.VMEM((B // 128, 8, 128), jnp.float32),
            pltpu.SemaphoreType.DMA,
            pltpu.SemaphoreType.DMA,
            pltpu.SemaphoreType.DMA,
            pltpu.SemaphoreType.DMA,
        ],
        compiler_params=pltpu.CompilerParams(use_tc_tiling_on_sc=False,
                                             needs_layout_passes=False),
    )
    out5 = k(idx_t, table_t)
    t = out5.transpose(2, 4, 0, 1, 3)
    return t.reshape(B, L, VOCAB)
